# baseline (speedup 1.0000x reference)
import jax
import jax.numpy as jnp
from jax.experimental import pallas as pl

N = 10000
D = 128
D2 = 256
T = 20
NUM_LAYERS = 3

M_BLK = 400
M_GRID = N // M_BLK


def _deg_body(c_ref, o_ref):
    s = jnp.sum(c_ref[...].astype(jnp.float32), axis=1, keepdims=True)
    o_ref[...] = jax.lax.rsqrt(jnp.maximum(s, 1.0))


def _prop_body(c_ref, zs_ref, dm_ref, accin_ref, invfact_ref,
               zsout_ref, accout_ref):
    t = jnp.dot(c_ref[...], zs_ref[...], preferred_element_type=jnp.float32)
    znew = t * dm_ref[...]
    zsout_ref[...] = (znew * dm_ref[...]).astype(jnp.bfloat16)
    accout_ref[...] = accin_ref[...] + znew * invfact_ref[...]


def _linear_body(h_ref, w_ref, d_ref, o_ref, os_ref):
    z = jnp.dot(h_ref[...], w_ref[...], preferred_element_type=jnp.float32)
    o_ref[...] = z
    os_ref[...] = (z * d_ref[...]).astype(jnp.bfloat16)


def _linear_relu_body(h_ref, w_ref, d_ref, o_ref, os_ref):
    z = jnp.dot(jnp.maximum(h_ref[...], 0.0), w_ref[...],
                preferred_element_type=jnp.float32)
    o_ref[...] = z
    os_ref[...] = (z * d_ref[...]).astype(jnp.bfloat16)


def _head_body(h_ref, w_ref, b_ref, o_ref):
    o_ref[...] = jnp.dot(jnp.maximum(h_ref[:, :D], 0.0), w_ref[...],
                         preferred_element_type=jnp.float32) + b_ref[...]


def _prop(C, Zs, dinv, acc, invfact):
    return pl.pallas_call(
        _prop_body,
        grid=(M_GRID,),
        in_specs=[
            pl.BlockSpec((M_BLK, N), lambda m: (m, 0)),
            pl.BlockSpec((N, D2), lambda m: (0, 0)),
            pl.BlockSpec((M_BLK, 1), lambda m: (m, 0)),
            pl.BlockSpec((M_BLK, D2), lambda m: (m, 0)),
            pl.BlockSpec((1, 1), lambda m: (0, 0)),
        ],
        out_specs=[
            pl.BlockSpec((M_BLK, D2), lambda m: (m, 0)),
            pl.BlockSpec((M_BLK, D2), lambda m: (m, 0)),
        ],
        out_shape=[
            jax.ShapeDtypeStruct((N, D2), jnp.bfloat16),
            jax.ShapeDtypeStruct((N, D2), jnp.float32),
        ],
    )(C, Zs, dinv, acc, invfact)


def _linear(h, W, dinv, relu):
    kin = h.shape[1]
    kout = W.shape[1]
    return pl.pallas_call(
        _linear_relu_body if relu else _linear_body,
        grid=(M_GRID,),
        in_specs=[
            pl.BlockSpec((M_BLK, kin), lambda m: (m, 0)),
            pl.BlockSpec((kin, kout), lambda m: (0, 0)),
            pl.BlockSpec((M_BLK, 1), lambda m: (m, 0)),
        ],
        out_specs=[
            pl.BlockSpec((M_BLK, kout), lambda m: (m, 0)),
            pl.BlockSpec((M_BLK, kout), lambda m: (m, 0)),
        ],
        out_shape=[
            jax.ShapeDtypeStruct((N, kout), jnp.float32),
            jax.ShapeDtypeStruct((N, kout), jnp.bfloat16),
        ],
    )(h, W, dinv)


def _head(acc, Wo, bo):
    return pl.pallas_call(
        _head_body,
        grid=(M_GRID,),
        in_specs=[
            pl.BlockSpec((M_BLK, D2), lambda m: (m, 0)),
            pl.BlockSpec((D, D), lambda m: (0, 0)),
            pl.BlockSpec((1, D), lambda m: (0, 0)),
        ],
        out_specs=pl.BlockSpec((M_BLK, D), lambda m: (m, 0)),
        out_shape=jax.ShapeDtypeStruct((N, D), jnp.float32),
    )(acc, Wo, bo.reshape(1, D))


def kernel(x, edge_index, Wr, Wi, Wo, bo):
    src = edge_index[0].astype(jnp.int32)
    dst = edge_index[1].astype(jnp.int32)
    lin = jnp.sort(dst * N + src)
    ones = jnp.ones((lin.shape[0],), jnp.bfloat16)
    C = (jnp.zeros((N * N,), jnp.bfloat16)
         .at[lin].add(ones, indices_are_sorted=True)
         .reshape(N, N))

    dinv = pl.pallas_call(
        _deg_body,
        grid=(M_GRID,),
        in_specs=[pl.BlockSpec((M_BLK, N), lambda m: (m, 0))],
        out_specs=pl.BlockSpec((M_BLK, 1), lambda m: (m, 0)),
        out_shape=jax.ShapeDtypeStruct((N, 1), jnp.float32),
    )(C)

    acc = None
    for l in range(NUM_LAYERS):
        if l == 0:
            W0 = jnp.concatenate([Wr[0], Wi[0]], axis=1)
            acc, Zs = _linear(x, W0, dinv, relu=False)
        else:
            Wc = jnp.concatenate([
                jnp.concatenate([Wr[l], Wi[l]], axis=1),
                jnp.concatenate([-Wi[l], Wr[l]], axis=1),
            ], axis=0)
            acc, Zs = _linear(acc, Wc, dinv, relu=True)
        fact = 1.0
        for t in range(1, T):
            fact = fact * t
            invfact = jnp.full((1, 1), 1.0 / fact, jnp.float32)
            Zs, acc = _prop(C, Zs, dinv, acc, invfact)

    return _head(acc, Wo, bo)


# speedup vs baseline: 1.0610x; 1.0610x over previous
import math

import jax
import jax.numpy as jnp
from jax.experimental import pallas as pl
from jax.experimental.pallas import tpu as pltpu

N = 10000
D = 128
D2 = 256
T = 20
NUM_LAYERS = 3

M_BLK = 400
M_GRID = N // M_BLK


def _deg_body(c_ref, o_ref):
    s = jnp.sum(c_ref[...].astype(jnp.float32), axis=1, keepdims=True)
    o_ref[...] = jax.lax.rsqrt(jnp.maximum(s, 1.0))


def _taylor_body(c_ref, zs0_ref, dm_ref, acc0_ref, invf_ref, accout_ref,
                 zs_a, zs_b, acc_s):
    t = pl.program_id(0)
    m = pl.program_id(1)
    rows = pl.ds(m * M_BLK, M_BLK)

    @pl.when(jnp.logical_and(t == 0, m == 0))
    def _():
        zs_a[...] = zs0_ref[...]

    def step(src_ref):
        zn = jnp.dot(c_ref[...], src_ref[...],
                     preferred_element_type=jnp.float32) * dm_ref[...]
        upd = zn * invf_ref[0, 0, 0]

        @pl.when(t == 0)
        def _():
            acc_s[rows, :] = acc0_ref[...] + upd

        @pl.when(t > 0)
        def _():
            acc_s[rows, :] = acc_s[rows, :] + upd

        return (zn * dm_ref[...]).astype(jnp.bfloat16)

    @pl.when(t % 2 == 0)
    def _():
        zs_b[rows, :] = step(zs_a)

    @pl.when(t % 2 == 1)
    def _():
        zs_a[rows, :] = step(zs_b)

    @pl.when(t == T - 2)
    def _():
        accout_ref[...] = acc_s[rows, :]


def _linear_body(h_ref, w_ref, d_ref, o_ref, os_ref):
    z = jnp.dot(h_ref[...], w_ref[...], preferred_element_type=jnp.float32)
    o_ref[...] = z
    os_ref[...] = (z * d_ref[...]).astype(jnp.bfloat16)


def _linear_relu_body(h_ref, w_ref, d_ref, o_ref, os_ref):
    z = jnp.dot(jnp.maximum(h_ref[...], 0.0), w_ref[...],
                preferred_element_type=jnp.float32)
    o_ref[...] = z
    os_ref[...] = (z * d_ref[...]).astype(jnp.bfloat16)


def _head_body(h_ref, w_ref, b_ref, o_ref):
    o_ref[...] = jnp.dot(jnp.maximum(h_ref[:, :D], 0.0), w_ref[...],
                         preferred_element_type=jnp.float32) + b_ref[...]


def _taylor(C, Zs, dinv, acc, invf):
    return pl.pallas_call(
        _taylor_body,
        grid=(T - 1, M_GRID),
        in_specs=[
            pl.BlockSpec((M_BLK, N), lambda t, m: (m, 0)),
            pl.BlockSpec((N, D2), lambda t, m: (0, 0)),
            pl.BlockSpec((M_BLK, 1), lambda t, m: (m, 0)),
            pl.BlockSpec((M_BLK, D2), lambda t, m: (m, 0)),
            pl.BlockSpec((1, 1, 1), lambda t, m: (t, 0, 0)),
        ],
        out_specs=pl.BlockSpec((M_BLK, D2), lambda t, m: (m, 0)),
        out_shape=jax.ShapeDtypeStruct((N, D2), jnp.float32),
        scratch_shapes=[
            pltpu.VMEM((N, D2), jnp.bfloat16),
            pltpu.VMEM((N, D2), jnp.bfloat16),
            pltpu.VMEM((N, D2), jnp.float32),
        ],
    )(C, Zs, dinv, acc, invf)


def _linear(h, W, dinv, relu):
    kin = h.shape[1]
    kout = W.shape[1]
    return pl.pallas_call(
        _linear_relu_body if relu else _linear_body,
        grid=(M_GRID,),
        in_specs=[
            pl.BlockSpec((M_BLK, kin), lambda m: (m, 0)),
            pl.BlockSpec((kin, kout), lambda m: (0, 0)),
            pl.BlockSpec((M_BLK, 1), lambda m: (m, 0)),
        ],
        out_specs=[
            pl.BlockSpec((M_BLK, kout), lambda m: (m, 0)),
            pl.BlockSpec((M_BLK, kout), lambda m: (m, 0)),
        ],
        out_shape=[
            jax.ShapeDtypeStruct((N, kout), jnp.float32),
            jax.ShapeDtypeStruct((N, kout), jnp.bfloat16),
        ],
    )(h, W, dinv)


def _head(acc, Wo, bo):
    return pl.pallas_call(
        _head_body,
        grid=(M_GRID,),
        in_specs=[
            pl.BlockSpec((M_BLK, D2), lambda m: (m, 0)),
            pl.BlockSpec((D, D), lambda m: (0, 0)),
            pl.BlockSpec((1, D), lambda m: (0, 0)),
        ],
        out_specs=pl.BlockSpec((M_BLK, D), lambda m: (m, 0)),
        out_shape=jax.ShapeDtypeStruct((N, D), jnp.float32),
    )(acc, Wo, bo.reshape(1, D))


def kernel(x, edge_index, Wr, Wi, Wo, bo):
    src = edge_index[0].astype(jnp.int32)
    dst = edge_index[1].astype(jnp.int32)
    C = jnp.zeros((N, N), jnp.bfloat16).at[dst, src].add(jnp.bfloat16(1.0))
    invf = jnp.array(
        [1.0 / math.factorial(t) for t in range(1, T)],
        jnp.float32).reshape(T - 1, 1, 1)

    dinv = pl.pallas_call(
        _deg_body,
        grid=(M_GRID,),
        in_specs=[pl.BlockSpec((M_BLK, N), lambda m: (m, 0))],
        out_specs=pl.BlockSpec((M_BLK, 1), lambda m: (m, 0)),
        out_shape=jax.ShapeDtypeStruct((N, 1), jnp.float32),
    )(C)

    acc = None
    for l in range(NUM_LAYERS):
        if l == 0:
            W0 = jnp.concatenate([Wr[0], Wi[0]], axis=1)
            acc, Zs = _linear(x, W0, dinv, relu=False)
        else:
            Wc = jnp.concatenate([
                jnp.concatenate([Wr[l], Wi[l]], axis=1),
                jnp.concatenate([-Wi[l], Wr[l]], axis=1),
            ], axis=0)
            acc, Zs = _linear(acc, Wc, dinv, relu=True)
        acc = _taylor(C, Zs, dinv, acc, invf)

    return _head(acc, Wo, bo)
